# baseline (speedup 1.0000x reference)
import jax
import jax.numpy as jnp
from jax.experimental import pallas as pl
from jax.experimental.pallas import tpu as pltpu


def _tc_lomid_body(x_ref, lo_ref, mid_ref):
    lo_ref[...] = x_ref[:, 0:48, :]
    mid_ref[...] = x_ref[:, 48:96, :]


def _tc_hi_body(x_ref, hi_ref):
    hi_ref[...] = x_ref[...]


def kernel(x, idx_low, idx_mid, idx_high):
    B, _, R, C = x.shape
    x3 = x.reshape(B, R, C)

    lo, mid = pl.pallas_call(
        _tc_lomid_body,
        grid=(B,),
        in_specs=[pl.BlockSpec((1, 96, C), lambda b: (b, 0, 0))],
        out_specs=(
            pl.BlockSpec((1, 48, C), lambda b: (b, 0, 0)),
            pl.BlockSpec((1, 48, C), lambda b: (b, 0, 0)),
        ),
        out_shape=(
            jax.ShapeDtypeStruct((B, 48, C), x.dtype),
            jax.ShapeDtypeStruct((B, 48, C), x.dtype),
        ),
    )(x3)

    hi = pl.pallas_call(
        _tc_hi_body,
        grid=(B,),
        in_specs=[pl.BlockSpec((1, 32, C), lambda b: (b, 3, 0))],
        out_specs=pl.BlockSpec((1, 32, C), lambda b: (b, 0, 0)),
        out_shape=jax.ShapeDtypeStruct((B, 32, C), x.dtype),
    )(x3)

    return lo.reshape(B, 1, 48, C), mid, hi.reshape(B, 1, 32, C)


# speedup vs baseline: 1.4928x; 1.4928x over previous
import jax
import jax.numpy as jnp
from jax.experimental import pallas as pl

_BB = 4


def _split_body(x_ref, lo_ref, mid_ref, hi_ref):
    lo_ref[...] = x_ref[:, 0:48, :]
    mid_ref[...] = x_ref[:, 48:96, :]
    hi_ref[...] = x_ref[:, 96:128, :]


def kernel(x, idx_low, idx_mid, idx_high):
    B, _, R, C = x.shape
    x3 = x.reshape(B, R, C)
    lo, mid, hi = pl.pallas_call(
        _split_body,
        grid=(B // _BB,),
        in_specs=[pl.BlockSpec((_BB, R, C), lambda b: (b, 0, 0))],
        out_specs=(
            pl.BlockSpec((_BB, 48, C), lambda b: (b, 0, 0)),
            pl.BlockSpec((_BB, 48, C), lambda b: (b, 0, 0)),
            pl.BlockSpec((_BB, 32, C), lambda b: (b, 0, 0)),
        ),
        out_shape=(
            jax.ShapeDtypeStruct((B, 48, C), x.dtype),
            jax.ShapeDtypeStruct((B, 48, C), x.dtype),
            jax.ShapeDtypeStruct((B, 32, C), x.dtype),
        ),
    )(x3)
    return lo.reshape(B, 1, 48, C), mid, hi.reshape(B, 1, 32, C)
